# baseline (speedup 1.0000x reference)
import functools

import jax
import jax.numpy as jnp
from jax import lax
from jax.experimental import pallas as pl
from jax.experimental.pallas import tpu as pltpu
from jax.experimental.pallas import tpu_sc as plsc

N = 524288
D = 256
C = 19
CR = 24
NW = 32
NS = 16
CHUNK = 128
LANES = 16
GRP = D // LANES

NPAIR = 22
SC_ROWS_PER_TILE = NPAIR * 2 * CHUNK
SC_ROWS = SC_ROWS_PER_TILE * NW
TC_ROWS = N - SC_ROWS
TC_BLOCK = 1024
TC_NBLK = TC_ROWS // TC_BLOCK

MOM = 0.9
W_NEW = 1.0 - MOM


def _sc_body(feat_hbm, lab_hbm, sums_hbm,
             fbuf0, fbuf1, lbv0, lbv1, acc, accb,
             fsem0, fsem1, lsem0, lsem1):
  cid = lax.axis_index("c")
  sid = lax.axis_index("s")
  wid = sid * 2 + cid
  base = wid * SC_ROWS_PER_TILE

  zeros = jnp.zeros((LANES,), jnp.float32)
  def _zrow(i, _):
    for j in range(GRP):
      acc[i, pl.ds(j * LANES, LANES)] = zeros
      accb[i, pl.ds(j * LANES, LANES)] = zeros
    return 0
  lax.fori_loop(0, CR, _zrow, 0)

  def start(c, fbuf, lbv, fsem, lsem):
    row0 = base + c * CHUNK
    pltpu.async_copy(feat_hbm.at[pl.ds(row0, CHUNK)], fbuf, fsem)
    pltpu.async_copy(lab_hbm.at[pl.ds(row0, CHUNK)], lbv, lsem)

  def wait(c, fbuf, lbv, fsem, lsem):
    row0 = base + c * CHUNK
    pltpu.make_async_copy(feat_hbm.at[pl.ds(row0, CHUNK)], fbuf, fsem).wait()
    pltpu.make_async_copy(lab_hbm.at[pl.ds(row0, CHUNK)], lbv, lsem).wait()

  def process(fbuf, lbuf):
    @plsc.parallel_loop(0, CHUNK // LANES)
    def _grp(g):
      lv = lbuf[pl.ds(g * LANES, LANES)]
      lbls = [lv[k] for k in range(LANES)]

      def loads(k):
        r = g * LANES + k
        return [fbuf[r, pl.ds(j * LANES, LANES)] for j in range(GRP)]

      def stores(k, vs):
        for j in range(GRP):
          dst = acc if j % 2 == 0 else accb
          plsc.addupdate(dst.at[lbls[k], pl.ds(j * LANES, LANES)], vs[j])

      pending = loads(0)
      for k in range(1, LANES):
        nxt = loads(k)
        stores(k - 1, pending)
        pending = nxt
      stores(LANES - 1, pending)

  start(0, fbuf0, lbv0, fsem0, lsem0)

  def pair(i, _):
    c0 = 2 * i
    start(c0 + 1, fbuf1, lbv1, fsem1, lsem1)
    wait(c0, fbuf0, lbv0, fsem0, lsem0)
    process(fbuf0, lbv0)

    @pl.when(i < NPAIR - 1)
    def _():
      start(c0 + 2, fbuf0, lbv0, fsem0, lsem0)

    wait(c0 + 1, fbuf1, lbv1, fsem1, lsem1)
    process(fbuf1, lbv1)
    return 0

  lax.fori_loop(0, NPAIR, pair, 0)

  def _merge(i, _):
    for j in range(GRP):
      acc[i, pl.ds(j * LANES, LANES)] = (
          acc[i, pl.ds(j * LANES, LANES)] + accb[i, pl.ds(j * LANES, LANES)])
    return 0
  lax.fori_loop(0, CR, _merge, 0)

  pltpu.sync_copy(acc, sums_hbm.at[wid])


@functools.cache
def _sc_partials():
  return pl.kernel(
      _sc_body,
      out_type=jax.ShapeDtypeStruct((NW, CR, D), jnp.float32),
      mesh=plsc.VectorSubcoreMesh(core_axis_name="c", subcore_axis_name="s",
                                  num_cores=2, num_subcores=NS),
      scratch_types=[
        pltpu.VMEM((CHUNK, D), jnp.float32),
        pltpu.VMEM((CHUNK, D), jnp.float32),
        pltpu.VMEM((CHUNK,), jnp.int32),
        pltpu.VMEM((CHUNK,), jnp.int32),
        pltpu.VMEM((CR, D), jnp.float32),
        pltpu.VMEM((CR, D), jnp.float32),
        pltpu.SemaphoreType.DMA,
        pltpu.SemaphoreType.DMA,
        pltpu.SemaphoreType.DMA,
        pltpu.SemaphoreType.DMA,
      ],
  )


def _tc_body(feat_ref, lab_ref, o_ref):
  lab = lab_ref[0, 0, :]
  iota = lax.broadcasted_iota(jnp.int32, (CR, TC_BLOCK), 0)
  oh = (iota == lab[None, :]).astype(jnp.float32)
  part = jnp.dot(oh, feat_ref[...], preferred_element_type=jnp.float32)

  @pl.when(pl.program_id(0) == 0)
  def _():
    o_ref[...] = jnp.zeros_like(o_ref)

  o_ref[...] += part


def _tc_sums(feats_tc, labs_tc):
  return pl.pallas_call(
      _tc_body,
      grid=(TC_NBLK,),
      in_specs=[
          pl.BlockSpec((TC_BLOCK, D), lambda i: (i, 0)),
          pl.BlockSpec((1, 1, TC_BLOCK), lambda i: (i, 0, 0)),
      ],
      out_specs=pl.BlockSpec((CR, D), lambda i: (0, 0)),
      out_shape=jax.ShapeDtypeStruct((CR, D), jnp.float32),
  )(feats_tc, labs_tc.reshape(TC_NBLK, 1, TC_BLOCK))


def _combine_body(sums_ref, tc_ref, lab_ref, proto_ref, o_ref):
  sums = (jnp.sum(sums_ref[...], axis=0) + tc_ref[...])[:C]
  labs = lab_ref[...]
  cnts = jnp.stack(
      [jnp.sum(jnp.where(labs == c, 1.0, 0.0)) for c in range(C)]
  )[:, None]
  mean = sums / jnp.maximum(cnts, 1.0)
  proto = proto_ref[...]
  o_ref[...] = jnp.where(cnts > 0.0, W_NEW * mean + MOM * proto, proto)


def kernel(features, labels, Proto):
  sums_sc = _sc_partials()(features, labels)
  sums_tc = _tc_sums(features[SC_ROWS:], labels[SC_ROWS:])
  labs2d = labels.reshape(N // 128, 128)
  return pl.pallas_call(
      _combine_body,
      out_shape=jax.ShapeDtypeStruct((C, D), jnp.float32),
  )(sums_sc, sums_tc, labs2d, Proto)


# speedup vs baseline: 1.3080x; 1.3080x over previous
import functools

import jax
import jax.numpy as jnp
from jax import lax
from jax.experimental import pallas as pl
from jax.experimental.pallas import tpu as pltpu
from jax.experimental.pallas import tpu_sc as plsc

N = 524288
D = 256
C = 19
CR = 24
NW = 32
NS = 16
CHUNK = 128
LANES = 16
GRP = D // LANES

NPAIR = 64
SC_ROWS_PER_TILE = NPAIR * 2 * CHUNK
SC_ROWS = SC_ROWS_PER_TILE * NW

MOM = 0.9
W_NEW = 1.0 - MOM


def _sc_body(feat_hbm, lab_hbm, sums_hbm,
             fbuf0, fbuf1, lbv0, lbv1, acc, accb,
             fsem0, fsem1, lsem0, lsem1):
  cid = lax.axis_index("c")
  sid = lax.axis_index("s")
  wid = sid * 2 + cid
  base = wid * SC_ROWS_PER_TILE

  zeros = jnp.zeros((LANES,), jnp.float32)
  def _zrow(i, _):
    for j in range(GRP):
      acc[i, pl.ds(j * LANES, LANES)] = zeros
      accb[i, pl.ds(j * LANES, LANES)] = zeros
    return 0
  lax.fori_loop(0, CR, _zrow, 0)

  def start(c, fbuf, lbv, fsem, lsem):
    row0 = base + c * CHUNK
    pltpu.async_copy(feat_hbm.at[pl.ds(row0, CHUNK)], fbuf, fsem)
    pltpu.async_copy(lab_hbm.at[pl.ds(row0, CHUNK)], lbv, lsem)

  def wait(c, fbuf, lbv, fsem, lsem):
    row0 = base + c * CHUNK
    pltpu.make_async_copy(feat_hbm.at[pl.ds(row0, CHUNK)], fbuf, fsem).wait()
    pltpu.make_async_copy(lab_hbm.at[pl.ds(row0, CHUNK)], lbv, lsem).wait()

  def process(fbuf, lbuf):
    @plsc.parallel_loop(0, CHUNK // LANES)
    def _grp(g):
      lv = lbuf[pl.ds(g * LANES, LANES)]
      lbls = [lv[k] for k in range(LANES)]

      def loads(k):
        r = g * LANES + k
        return [fbuf[r, pl.ds(j * LANES, LANES)] for j in range(GRP)]

      def stores(k, vs):
        for j in range(GRP):
          dst = acc if j % 2 == 0 else accb
          plsc.addupdate(dst.at[lbls[k], pl.ds(j * LANES, LANES)], vs[j])

      pending = loads(0)
      for k in range(1, LANES):
        nxt = loads(k)
        stores(k - 1, pending)
        pending = nxt
      stores(LANES - 1, pending)

  start(0, fbuf0, lbv0, fsem0, lsem0)

  def pair(i, _):
    c0 = 2 * i
    start(c0 + 1, fbuf1, lbv1, fsem1, lsem1)
    wait(c0, fbuf0, lbv0, fsem0, lsem0)
    process(fbuf0, lbv0)

    @pl.when(i < NPAIR - 1)
    def _():
      start(c0 + 2, fbuf0, lbv0, fsem0, lsem0)

    wait(c0 + 1, fbuf1, lbv1, fsem1, lsem1)
    process(fbuf1, lbv1)
    return 0

  lax.fori_loop(0, NPAIR, pair, 0)

  def _merge(i, _):
    for j in range(GRP):
      acc[i, pl.ds(j * LANES, LANES)] = (
          acc[i, pl.ds(j * LANES, LANES)] + accb[i, pl.ds(j * LANES, LANES)])
    return 0
  lax.fori_loop(0, CR, _merge, 0)

  pltpu.sync_copy(acc, sums_hbm.at[wid])


@functools.cache
def _sc_partials():
  return pl.kernel(
      _sc_body,
      out_type=jax.ShapeDtypeStruct((NW, CR, D), jnp.float32),
      mesh=plsc.VectorSubcoreMesh(core_axis_name="c", subcore_axis_name="s",
                                  num_cores=2, num_subcores=NS),
      scratch_types=[
        pltpu.VMEM((CHUNK, D), jnp.float32),
        pltpu.VMEM((CHUNK, D), jnp.float32),
        pltpu.VMEM((CHUNK,), jnp.int32),
        pltpu.VMEM((CHUNK,), jnp.int32),
        pltpu.VMEM((CR, D), jnp.float32),
        pltpu.VMEM((CR, D), jnp.float32),
        pltpu.SemaphoreType.DMA,
        pltpu.SemaphoreType.DMA,
        pltpu.SemaphoreType.DMA,
        pltpu.SemaphoreType.DMA,
      ],
  )


def _combine_body(sums_ref, lab_ref, proto_ref, o_ref):
  sums = jnp.sum(sums_ref[...], axis=0)[:C]
  labs = lab_ref[...]
  cnts = jnp.stack(
      [jnp.sum(jnp.where(labs == c, 1.0, 0.0)) for c in range(C)]
  )[:, None]
  mean = sums / jnp.maximum(cnts, 1.0)
  proto = proto_ref[...]
  o_ref[...] = jnp.where(cnts > 0.0, W_NEW * mean + MOM * proto, proto)


def kernel(features, labels, Proto):
  sums_sc = _sc_partials()(features, labels)
  labs2d = labels.reshape(N // 128, 128)
  return pl.pallas_call(
      _combine_body,
      out_shape=jax.ShapeDtypeStruct((C, D), jnp.float32),
  )(sums_sc, labs2d, Proto)


# speedup vs baseline: 1.3113x; 1.0025x over previous
import functools

import jax
import jax.numpy as jnp
from jax import lax
from jax.experimental import pallas as pl
from jax.experimental.pallas import tpu as pltpu
from jax.experimental.pallas import tpu_sc as plsc

N = 524288
D = 256
C = 19
CR = 24
NW = 32
NS = 16
CHUNK = 128
LANES = 16
GRP = D // LANES

NPAIR = 64
SC_ROWS_PER_TILE = NPAIR * 2 * CHUNK
SC_ROWS = SC_ROWS_PER_TILE * NW

MOM = 0.9
W_NEW = 1.0 - MOM


def _sc_body(feat_hbm, lab_hbm, sums_hbm,
             fbuf0, fbuf1, lbv0, lbv1, acc, accb, accc, accd,
             fsem0, fsem1, lsem0, lsem1):
  cid = lax.axis_index("c")
  sid = lax.axis_index("s")
  wid = sid * 2 + cid
  base = wid * SC_ROWS_PER_TILE

  zeros = jnp.zeros((LANES,), jnp.float32)
  def _zrow(i, _):
    for j in range(GRP):
      acc[i, pl.ds(j * LANES, LANES)] = zeros
      accb[i, pl.ds(j * LANES, LANES)] = zeros
      accc[i, pl.ds(j * LANES, LANES)] = zeros
      accd[i, pl.ds(j * LANES, LANES)] = zeros
    return 0
  lax.fori_loop(0, CR, _zrow, 0)

  def start(c, fbuf, lbv, fsem, lsem):
    row0 = base + c * CHUNK
    pltpu.async_copy(feat_hbm.at[pl.ds(row0, CHUNK)], fbuf, fsem)
    pltpu.async_copy(lab_hbm.at[pl.ds(row0, CHUNK)], lbv, lsem)

  def wait(c, fbuf, lbv, fsem, lsem):
    row0 = base + c * CHUNK
    pltpu.make_async_copy(feat_hbm.at[pl.ds(row0, CHUNK)], fbuf, fsem).wait()
    pltpu.make_async_copy(lab_hbm.at[pl.ds(row0, CHUNK)], lbv, lsem).wait()

  def process(fbuf, lbuf):
    @plsc.parallel_loop(0, CHUNK // LANES)
    def _grp(g):
      lv = lbuf[pl.ds(g * LANES, LANES)]
      lbls = [lv[k] for k in range(LANES)]

      def loads(k):
        r = g * LANES + k
        return [fbuf[r, pl.ds(j * LANES, LANES)] for j in range(GRP)]

      banks = [acc, accb, accc, accd]

      def stores(k, vs):
        for j in range(GRP):
          plsc.addupdate(banks[j % 4].at[lbls[k], pl.ds(j * LANES, LANES)],
                         vs[j])

      pending = loads(0)
      for k in range(1, LANES):
        nxt = loads(k)
        stores(k - 1, pending)
        pending = nxt
      stores(LANES - 1, pending)

  start(0, fbuf0, lbv0, fsem0, lsem0)

  def pair(i, _):
    c0 = 2 * i
    start(c0 + 1, fbuf1, lbv1, fsem1, lsem1)
    wait(c0, fbuf0, lbv0, fsem0, lsem0)
    process(fbuf0, lbv0)

    @pl.when(i < NPAIR - 1)
    def _():
      start(c0 + 2, fbuf0, lbv0, fsem0, lsem0)

    wait(c0 + 1, fbuf1, lbv1, fsem1, lsem1)
    process(fbuf1, lbv1)
    return 0

  lax.fori_loop(0, NPAIR, pair, 0)

  def _merge(i, _):
    for j in range(GRP):
      acc[i, pl.ds(j * LANES, LANES)] = (
          (acc[i, pl.ds(j * LANES, LANES)] + accb[i, pl.ds(j * LANES, LANES)])
          + (accc[i, pl.ds(j * LANES, LANES)]
             + accd[i, pl.ds(j * LANES, LANES)]))
    return 0
  lax.fori_loop(0, CR, _merge, 0)

  pltpu.sync_copy(acc, sums_hbm.at[wid])


@functools.cache
def _sc_partials():
  return pl.kernel(
      _sc_body,
      out_type=jax.ShapeDtypeStruct((NW, CR, D), jnp.float32),
      mesh=plsc.VectorSubcoreMesh(core_axis_name="c", subcore_axis_name="s",
                                  num_cores=2, num_subcores=NS),
      scratch_types=[
        pltpu.VMEM((CHUNK, D), jnp.float32),
        pltpu.VMEM((CHUNK, D), jnp.float32),
        pltpu.VMEM((CHUNK,), jnp.int32),
        pltpu.VMEM((CHUNK,), jnp.int32),
        pltpu.VMEM((CR, D), jnp.float32),
        pltpu.VMEM((CR, D), jnp.float32),
        pltpu.VMEM((CR, D), jnp.float32),
        pltpu.VMEM((CR, D), jnp.float32),
        pltpu.SemaphoreType.DMA,
        pltpu.SemaphoreType.DMA,
        pltpu.SemaphoreType.DMA,
        pltpu.SemaphoreType.DMA,
      ],
  )


def _combine_body(sums_ref, lab_ref, proto_ref, o_ref):
  sums = jnp.sum(sums_ref[...], axis=0)[:C]
  labs = lab_ref[...]
  cnts = jnp.stack(
      [jnp.sum(jnp.where(labs == c, 1.0, 0.0)) for c in range(C)]
  )[:, None]
  mean = sums / jnp.maximum(cnts, 1.0)
  proto = proto_ref[...]
  o_ref[...] = jnp.where(cnts > 0.0, W_NEW * mean + MOM * proto, proto)


def kernel(features, labels, Proto):
  sums_sc = _sc_partials()(features, labels)
  labs2d = labels.reshape(N // 128, 128)
  return pl.pallas_call(
      _combine_body,
      out_shape=jax.ShapeDtypeStruct((C, D), jnp.float32),
  )(sums_sc, labs2d, Proto)
